# baseline (speedup 1.0000x reference)
import functools

import jax
import jax.numpy as jnp
from jax import lax
from jax.experimental import pallas as pl
from jax.experimental.pallas import tpu as pltpu
from jax.experimental.pallas import tpu_sc as plsc

N_LAYERS = 8
N_KV_HEADS = 8
HEAD_DIM = 128
MAX_ENTRIES = 1024
HIDDEN = 2048
SEQ = 2048

_NC = 2
_NS = 16
_NW = _NC * _NS

_TOTAL_ROWS = N_LAYERS * MAX_ENTRIES * N_KV_HEADS
_CHUNK = 128
_NSLOT = 4


def _gate_topk_body(sc_ref, sr_ref, out_ref):
    s_col = sc_ref[...]
    s_row = sr_ref[...]
    irow = lax.broadcasted_iota(jnp.int32, (SEQ, SEQ), 0)
    jlane = lax.broadcasted_iota(jnp.int32, (SEQ, SEQ), 1)
    gt = s_row > s_col
    tie = (s_row == s_col) & (jlane < irow)
    cnt = jnp.where(gt | tie, jnp.ones((SEQ, SEQ), jnp.float32),
                    jnp.zeros((SEQ, SEQ), jnp.float32))
    rank_col = jnp.sum(cnt, axis=1, keepdims=True).astype(jnp.int32)
    i_sub = lax.broadcasted_iota(jnp.int32, (SEQ, HEAD_DIM), 0)
    o_lane = lax.broadcasted_iota(jnp.int32, (SEQ, HEAD_DIM), 1)
    zero = jnp.zeros((SEQ, HEAD_DIM), jnp.int32)
    for a in range(MAX_ENTRIES // HEAD_DIM):
        sel = jnp.where(rank_col == a * HEAD_DIM + o_lane, i_sub, zero)
        out_ref[a:a + 1, :] = jnp.sum(sel, axis=0, keepdims=True)


def _gate_topk(scores):
    return pl.pallas_call(
        _gate_topk_body,
        out_shape=jax.ShapeDtypeStruct((MAX_ENTRIES // HEAD_DIM, HEAD_DIM),
                                       jnp.int32),
    )(scores.reshape(SEQ, 1), scores.reshape(1, SEQ))


def _sc_gather_body(tidx_hbm, ktab_hbm, vtab_hbm, kout_hbm, vout_hbm,
                    tidx_v, idx_v, *rest):
    wid = lax.axis_index("s") * _NC + lax.axis_index("c")
    table = wid % 2
    p_base = (wid // 2) * 4

    pltpu.sync_copy(tidx_hbm, tidx_v)

    def build(v, carry):
        t = tidx_v[pl.ds(v * 16, 16)]
        for q in range(4):
            idx_v[q, pl.ds(v * 16, 16)] = (p_base + q) * SEQ + t
        return carry

    lax.fori_loop(0, MAX_ENTRIES // 16, build, 0)

    B = _NSLOT
    bufs = rest[0:B]
    gsems = rest[B:2 * B]
    wsems = rest[2 * B:3 * B]

    gh = {}
    wh = {}
    n_rchunk = MAX_ENTRIES // _CHUNK

    def gather(c, src_hbm):
        s = c % B
        q = c // n_rchunk
        isl = idx_v.at[q, pl.ds((c % n_rchunk) * _CHUNK, _CHUNK)]
        gh[c] = pltpu.async_copy(src_hbm.at[isl], bufs[s], gsems[s])

    def write(c, dst_hbm):
        s = c % B
        q = c // n_rchunk
        p = p_base + q
        layer = p // N_KV_HEADS
        h = p % N_KV_HEADS
        dst = (layer, pl.ds((c % n_rchunk) * _CHUNK, _CHUNK), h)
        wh[c] = pltpu.async_copy(bufs[s], dst_hbm.at[dst], wsems[s])

    nchunk = 4 * n_rchunk

    def run(src_hbm, dst_hbm):
        for c in range(B - 1):
            gather(c, src_hbm)
        for c in range(nchunk):
            gh.pop(c).wait()
            write(c, dst_hbm)
            n = c + B - 1
            if n < nchunk:
                if c >= 1:
                    wh.pop(c - 1).wait()
                gather(n, src_hbm)
        for c in sorted(wh):
            wh.pop(c).wait()

    @pl.when(table == 0)
    def _():
        run(ktab_hbm, kout_hbm)

    @pl.when(table == 1)
    def _():
        run(vtab_hbm, vout_hbm)


@functools.lru_cache(maxsize=1)
def _make_sc_gather():
    return functools.partial(
        pl.kernel,
        mesh=plsc.VectorSubcoreMesh(core_axis_name="c", subcore_axis_name="s"),
        compiler_params=pltpu.CompilerParams(needs_layout_passes=False),
        out_type=[
            jax.ShapeDtypeStruct((N_LAYERS, MAX_ENTRIES, N_KV_HEADS, HEAD_DIM),
                                 jnp.float32),
            jax.ShapeDtypeStruct((N_LAYERS, MAX_ENTRIES, N_KV_HEADS, HEAD_DIM),
                                 jnp.float32),
        ],
        scratch_types=[
            pltpu.VMEM((MAX_ENTRIES,), jnp.int32),
            pltpu.VMEM((4, MAX_ENTRIES), jnp.int32),
        ] + [pltpu.VMEM((_CHUNK, HEAD_DIM), jnp.float32)] * _NSLOT
          + [pltpu.SemaphoreType.DMA] * (2 * _NSLOT),
    )(_sc_gather_body)


@jax.jit
def kernel(hidden_states, kv_keys, kv_values, keys_buf, values_buf,
           gate_w, gate_b):
    del keys_buf, values_buf
    logits = jnp.einsum('bsh,oh->bso', hidden_states, gate_w) + gate_b
    gate_scores = jax.nn.sigmoid(logits)[0, :, 0]
    tidx = _gate_topk(gate_scores).reshape(MAX_ENTRIES)
    ktab = kv_keys.reshape(N_LAYERS * N_KV_HEADS * SEQ, HEAD_DIM)
    vtab = kv_values.reshape(N_LAYERS * N_KV_HEADS * SEQ, HEAD_DIM)
    new_k, new_v = _make_sc_gather()(tidx, ktab, vtab)
    return new_k, new_v


# speedup vs baseline: 1.0086x; 1.0086x over previous
import functools

import jax
import jax.numpy as jnp
from jax import lax
from jax.experimental import pallas as pl
from jax.experimental.pallas import tpu as pltpu
from jax.experimental.pallas import tpu_sc as plsc

N_LAYERS = 8
N_KV_HEADS = 8
HEAD_DIM = 128
MAX_ENTRIES = 1024
HIDDEN = 2048
SEQ = 2048

_NC = 2
_CHUNK = 64
_NSLOT = 4


def _gate_topk_body(sc_ref, sr_ref, out_ref):
    s_col = sc_ref[...]
    s_row = sr_ref[...]
    irow = lax.broadcasted_iota(jnp.int32, (SEQ, SEQ), 0)
    jlane = lax.broadcasted_iota(jnp.int32, (SEQ, SEQ), 1)
    gt = s_row > s_col
    tie = (s_row == s_col) & (jlane < irow)
    cnt = jnp.where(gt | tie, jnp.ones((SEQ, SEQ), jnp.float32),
                    jnp.zeros((SEQ, SEQ), jnp.float32))
    rank_col = jnp.sum(cnt, axis=1, keepdims=True).astype(jnp.int32)
    i_sub = lax.broadcasted_iota(jnp.int32, (SEQ, HEAD_DIM), 0)
    o_lane = lax.broadcasted_iota(jnp.int32, (SEQ, HEAD_DIM), 1)
    zero = jnp.zeros((SEQ, HEAD_DIM), jnp.int32)
    for a in range(MAX_ENTRIES // HEAD_DIM):
        sel = jnp.where(rank_col == a * HEAD_DIM + o_lane, i_sub, zero)
        out_ref[a:a + 1, :] = jnp.sum(sel, axis=0, keepdims=True)


def _gate_topk(scores):
    return pl.pallas_call(
        _gate_topk_body,
        out_shape=jax.ShapeDtypeStruct((MAX_ENTRIES // HEAD_DIM, HEAD_DIM),
                                       jnp.int32),
    )(scores.reshape(SEQ, 1), scores.reshape(1, SEQ))


def _sc_gather_body(tidx_hbm, ktab_hbm, vtab_hbm, kout_hbm, vout_hbm,
                    tidx_v, idx_v, *rest):
    wid = lax.axis_index("s") * _NC + lax.axis_index("c")
    p0 = wid * 2

    pltpu.sync_copy(tidx_hbm, tidx_v)

    def build(v, carry):
        t = tidx_v[pl.ds(v * 16, 16)]
        idx_v[0, pl.ds(v * 16, 16)] = p0 * SEQ + t
        idx_v[1, pl.ds(v * 16, 16)] = (p0 + 1) * SEQ + t
        return carry

    lax.fori_loop(0, MAX_ENTRIES // 16, build, 0)

    B = _NSLOT
    kbufs = rest[0:B]
    vbufs = rest[B:2 * B]
    gksems = rest[2 * B:3 * B]
    gvsems = rest[3 * B:4 * B]
    wksems = rest[4 * B:5 * B]
    wvsems = rest[5 * B:6 * B]

    gh = {}
    wh = {}
    n_rchunk = MAX_ENTRIES // _CHUNK

    def gather(c):
        s = c % B
        q = c // n_rchunk
        isl = idx_v.at[q, pl.ds((c % n_rchunk) * _CHUNK, _CHUNK)]
        gh[c] = (pltpu.async_copy(ktab_hbm.at[isl], kbufs[s], gksems[s]),
                 pltpu.async_copy(vtab_hbm.at[isl], vbufs[s], gvsems[s]))

    def write(c):
        s = c % B
        q = c // n_rchunk
        p = p0 + q
        layer = p // N_KV_HEADS
        h = p % N_KV_HEADS
        dst = (layer, pl.ds((c % n_rchunk) * _CHUNK, _CHUNK), h)
        wh[c] = (pltpu.async_copy(kbufs[s], kout_hbm.at[dst], wksems[s]),
                 pltpu.async_copy(vbufs[s], vout_hbm.at[dst], wvsems[s]))

    nchunk = 2 * n_rchunk
    for c in range(B - 1):
        gather(c)
    for c in range(nchunk):
        for cp in gh.pop(c):
            cp.wait()
        write(c)
        n = c + B - 1
        if n < nchunk:
            if c >= 1:
                for cp in wh.pop(c - 1):
                    cp.wait()
            gather(n)
    for c in sorted(wh):
        for cp in wh.pop(c):
            cp.wait()


@functools.lru_cache(maxsize=1)
def _make_sc_gather():
    return functools.partial(
        pl.kernel,
        mesh=plsc.VectorSubcoreMesh(core_axis_name="c", subcore_axis_name="s"),
        compiler_params=pltpu.CompilerParams(needs_layout_passes=False),
        out_type=[
            jax.ShapeDtypeStruct((N_LAYERS, MAX_ENTRIES, N_KV_HEADS, HEAD_DIM),
                                 jnp.float32),
            jax.ShapeDtypeStruct((N_LAYERS, MAX_ENTRIES, N_KV_HEADS, HEAD_DIM),
                                 jnp.float32),
        ],
        scratch_types=[
            pltpu.VMEM((MAX_ENTRIES,), jnp.int32),
            pltpu.VMEM((2, MAX_ENTRIES), jnp.int32),
        ] + [pltpu.VMEM((_CHUNK, HEAD_DIM), jnp.float32)] * (2 * _NSLOT)
          + [pltpu.SemaphoreType.DMA] * (4 * _NSLOT),
    )(_sc_gather_body)


@jax.jit
def kernel(hidden_states, kv_keys, kv_values, keys_buf, values_buf,
           gate_w, gate_b):
    del keys_buf, values_buf
    logits = jnp.einsum('bsh,oh->bso', hidden_states, gate_w) + gate_b
    gate_scores = jax.nn.sigmoid(logits)[0, :, 0]
    tidx = _gate_topk(gate_scores).reshape(MAX_ENTRIES)
    ktab = kv_keys.reshape(N_LAYERS * N_KV_HEADS * SEQ, HEAD_DIM)
    vtab = kv_values.reshape(N_LAYERS * N_KV_HEADS * SEQ, HEAD_DIM)
    new_k, new_v = _make_sc_gather()(tidx, ktab, vtab)
    return new_k, new_v
